# baseline (speedup 1.0000x reference)
import functools

import jax
import jax.numpy as jnp
from jax import lax
from jax.experimental import pallas as pl
from jax.experimental.pallas import tpu as pltpu
from jax.experimental.pallas import tpu_sc as plsc

N = 10000
E = 320000
D = 128
EMB = 128
F1 = D + EMB
CC = 128
NC = 2

NUM_CORES = 2
NUM_SUB = 16
NW = NUM_CORES * NUM_SUB
EPT = E // NW
K = 128
KB = 39
BLK_E = KB * K
KT = EPT - 2 * BLK_E
NP = 10240
ROWS_PER_TILE = NP // NUM_SUB
ZR = 160


@functools.lru_cache(maxsize=1)
def _build_sc_segment_sum():
    mesh = plsc.VectorSubcoreMesh(core_axis_name="c", subcore_axis_name="s",
                                  num_cores=NUM_CORES, num_subcores=NUM_SUB)

    @functools.partial(
        pl.kernel,
        out_type=jax.ShapeDtypeStruct((NUM_CORES, NP, D), jnp.float32),
        mesh=mesh,
        scratch_types=[
            pltpu.VMEM((BLK_E + KT,), jnp.int32),
            pltpu.VMEM((BLK_E + KT,), jnp.int32),
            pltpu.VMEM((K, D), jnp.float32),
            pltpu.VMEM((K, D), jnp.float32),
            pltpu.SemaphoreType.DMA,
            pltpu.SemaphoreType.DMA,
            pltpu.SemaphoreType.DMA,
            pltpu.SemaphoreType.DMA,
            pltpu.SemaphoreType.DMA,
            pltpu.VMEM_SHARED((NP, D), jnp.float32),
        ],
        compiler_params=pltpu.CompilerParams(use_tc_tiling_on_sc=False),
    )
    def _sc_segment_sum(hw_hbm, src_hbm, dst_hbm, zeros_hbm, out_hbm,
                        src_v, dst_v, rows0_v, rows1_v,
                        gsem0, gsem1, ssem0, ssem1, zsem, acc):
        c = lax.axis_index("c")
        s = lax.axis_index("s")
        w = s * NUM_CORES + c
        rows0 = s * ROWS_PER_TILE

        def sidx(j):
            return src_v.at[pl.ds(j * K, K)]

        def didx(j):
            return dst_v.at[pl.ds(j * K, K)]

        def gwait(rows_v, gsem, j):
            pltpu.make_async_copy(hw_hbm.at[sidx(j)], rows_v, gsem).wait()

        def sstart(rows_v, ssem, j):
            pltpu.async_copy(rows_v, acc.at[didx(j)], ssem, add=True)

        def swait(rows_v, ssem):
            pltpu.make_async_copy(rows_v, acc.at[didx(0)], ssem).wait()

        def scat(rows_v, j):
            pltpu.sync_copy(rows_v, acc.at[didx(j)], add=True)

        def body(jj, carry):
            j = jj * 2
            gwait(rows0_v, gsem0, j)
            sstart(rows0_v, ssem0, j)
            gwait(rows1_v, gsem1, j + 1)
            sstart(rows1_v, ssem1, j + 1)
            swait(rows0_v, ssem0)
            pltpu.async_copy(hw_hbm.at[sidx(j + 2)], rows0_v, gsem0)
            swait(rows1_v, ssem1)
            pltpu.async_copy(hw_hbm.at[sidx(j + 3)], rows1_v, gsem1)
            return carry

        for i in range(ROWS_PER_TILE // ZR):
            pltpu.async_copy(zeros_hbm, acc.at[pl.ds(rows0 + i * ZR, ZR)],
                             zsem)
        base = w * EPT
        pltpu.sync_copy(src_hbm.at[pl.ds(base, BLK_E)],
                        src_v.at[pl.ds(0, BLK_E)])
        pltpu.sync_copy(dst_hbm.at[pl.ds(base, BLK_E)],
                        dst_v.at[pl.ds(0, BLK_E)])
        pltpu.async_copy(hw_hbm.at[sidx(0)], rows0_v, gsem0)
        pltpu.async_copy(hw_hbm.at[sidx(1)], rows1_v, gsem1)
        for i in range(ROWS_PER_TILE // ZR):
            pltpu.make_async_copy(zeros_hbm, acc.at[pl.ds(rows0, ZR)],
                                  zsem).wait()
        plsc.subcore_barrier()

        lax.fori_loop(0, (KB - 3) // 2, body, 0)
        gwait(rows0_v, gsem0, KB - 3)
        scat(rows0_v, KB - 3)
        pltpu.async_copy(hw_hbm.at[sidx(KB - 1)], rows0_v, gsem0)
        gwait(rows1_v, gsem1, KB - 2)
        scat(rows1_v, KB - 2)
        gwait(rows0_v, gsem0, KB - 1)
        scat(rows0_v, KB - 1)

        pltpu.sync_copy(src_hbm.at[pl.ds(base + BLK_E, BLK_E + KT)], src_v)
        pltpu.sync_copy(dst_hbm.at[pl.ds(base + BLK_E, BLK_E + KT)], dst_v)
        pltpu.async_copy(hw_hbm.at[sidx(0)], rows0_v, gsem0)
        pltpu.async_copy(hw_hbm.at[sidx(1)], rows1_v, gsem1)
        lax.fori_loop(0, (KB - 3) // 2, body, 0)
        rt = rows1_v.at[pl.ds(0, KT)]
        sit = src_v.at[pl.ds(BLK_E, KT)]
        dit = dst_v.at[pl.ds(BLK_E, KT)]
        gwait(rows0_v, gsem0, KB - 3)
        scat(rows0_v, KB - 3)
        pltpu.async_copy(hw_hbm.at[sidx(KB - 1)], rows0_v, gsem0)
        gwait(rows1_v, gsem1, KB - 2)
        scat(rows1_v, KB - 2)
        pltpu.async_copy(hw_hbm.at[sit], rt, gsem1)
        gwait(rows0_v, gsem0, KB - 1)
        scat(rows0_v, KB - 1)
        pltpu.make_async_copy(hw_hbm.at[sit], rt, gsem1).wait()
        pltpu.sync_copy(rt, acc.at[dit], add=True)
        plsc.subcore_barrier()
        pltpu.sync_copy(acc.at[pl.ds(rows0, ROWS_PER_TILE)],
                        out_hbm.at[c, pl.ds(rows0, ROWS_PER_TILE)])

    def wrapped(hw, src, dst, zeros):
        return _sc_segment_sum(hw, src, dst, zeros)

    return wrapped


BR = 1000


def _mm_body(x_ref, w_ref, o_ref):
    o_ref[...] = jnp.dot(x_ref[...], w_ref[...],
                         preferred_element_type=jnp.float32)


_mm = pl.pallas_call(
    _mm_body,
    grid=(N // BR,),
    in_specs=[pl.BlockSpec((BR, D), lambda i: (i, 0)),
              pl.BlockSpec((D, D), lambda i: (0, 0))],
    out_specs=pl.BlockSpec((BR, D), lambda i: (i, 0)),
    out_shape=jax.ShapeDtypeStruct((N, D), jnp.float32),
)


def _gru_body(h_ref, p_ref, Wz_ref, Uz_ref, Wr_ref, Ur_ref,
              Wh_ref, Uh_ref, b_ref, Wn_ref, ho_ref, hwo_ref):
    h = h_ref[...]
    m = p_ref[0] + p_ref[1]
    dot = functools.partial(jnp.dot, preferred_element_type=jnp.float32)
    z = jax.nn.sigmoid(dot(m, Wz_ref[...]) + dot(h, Uz_ref[...]) + b_ref[0])
    r = jax.nn.sigmoid(dot(m, Wr_ref[...]) + dot(h, Ur_ref[...]) + b_ref[1])
    hc = jnp.tanh(dot(m, Wh_ref[...]) + dot(r * h, Uh_ref[...]) + b_ref[2])
    hn = (1.0 - z) * h + z * hc
    ho_ref[...] = hn
    hwo_ref[...] = dot(hn, Wn_ref[...])


_gru_step = pl.pallas_call(
    _gru_body,
    grid=(N // BR,),
    in_specs=[pl.BlockSpec((BR, D), lambda i: (i, 0)),
              pl.BlockSpec((2, BR, D), lambda i: (0, i, 0)),
              pl.BlockSpec((D, D), lambda i: (0, 0)),
              pl.BlockSpec((D, D), lambda i: (0, 0)),
              pl.BlockSpec((D, D), lambda i: (0, 0)),
              pl.BlockSpec((D, D), lambda i: (0, 0)),
              pl.BlockSpec((D, D), lambda i: (0, 0)),
              pl.BlockSpec((D, D), lambda i: (0, 0)),
              pl.BlockSpec((3, D), lambda i: (0, 0)),
              pl.BlockSpec((D, D), lambda i: (0, 0))],
    out_specs=[pl.BlockSpec((BR, D), lambda i: (i, 0)),
               pl.BlockSpec((BR, D), lambda i: (i, 0))],
    out_shape=[jax.ShapeDtypeStruct((N, D), jnp.float32),
               jax.ShapeDtypeStruct((N, D), jnp.float32)],
)


def _shift_down(a):
    return jnp.concatenate([jnp.zeros((1, a.shape[1]), a.dtype), a[:-1]], 0)


def _shift_up(a):
    return jnp.concatenate([a[1:], jnp.zeros((1, a.shape[1]), a.dtype)], 0)


def _conv3_in(a, W_ref, b_ref):
    dot = functools.partial(jnp.dot, preferred_element_type=jnp.float32)
    return (dot(_shift_down(a), W_ref[0]) + dot(a, W_ref[1])
            + dot(_shift_up(a), W_ref[2]) + b_ref[0])


def _pool3s2(c):
    r = c.reshape(c.shape[0] // 2, 2, c.shape[1])
    e = r[:, 0, :]
    o = r[:, 1, :]
    return jnp.maximum(jnp.maximum(e, o), _shift_down(o))


def _readout_body(h_ref, x_ref, Wa1_ref, ba1_ref, Wb1_ref, bb1_ref,
                  Wa2_ref, ba2_ref, Wb2_ref, bb2_ref,
                  wa_ref, fa_ref, wb_ref, fb_ref, o_ref):
    dot = functools.partial(jnp.dot, preferred_element_type=jnp.float32)
    h = h_ref[...]
    zc = jnp.concatenate([h, x_ref[...]], 1)
    pa = _pool3s2(jnp.maximum(_conv3_in(zc, Wa1_ref, ba1_ref), 0.0))
    pb = _pool3s2(jnp.maximum(_conv3_in(h, Wb1_ref, bb1_ref), 0.0))
    p2a = _pool3s2(jnp.maximum(_conv3_in(pa, Wa2_ref, ba2_ref), 0.0))
    p2b = _pool3s2(jnp.maximum(_conv3_in(pb, Wb2_ref, bb2_ref), 0.0))
    ya = dot(p2a, wa_ref[...]) + fa_ref[0]
    yb = dot(p2b, wb_ref[...]) + fb_ref[0]
    o_ref[...] = jnp.sum(ya * yb, axis=0, keepdims=True) * (1.0 / (N // 4))


_readout = pl.pallas_call(
    _readout_body,
    out_shape=jax.ShapeDtypeStruct((1, NC), jnp.float32),
    compiler_params=pltpu.CompilerParams(vmem_limit_bytes=100 * 1024 * 1024),
)


def kernel(x, ast_edge_index, cfg_edge_index, ddg_edge_index, ncs_edge_index,
           W_edge, Wz, Uz, bz, Wr, Ur, br, Wh, Uh, bh,
           Wa1, ba1, Wa2, ba2, fca_w, fca_b,
           Wb1, bb1, Wb2, bb2, fcb_w, fcb_b):
    edges = [ast_edge_index, cfg_edge_index, ddg_edge_index, ncs_edge_index]
    zeros_tile = jnp.zeros((ZR, D), jnp.float32)
    b3 = jnp.stack([bz, br, bh])

    sc_segment_sum = _build_sc_segment_sum()
    h = x
    hw = _mm(h, W_edge[0])
    for t in range(4):
        parts = sc_segment_sum(hw, edges[t][0], edges[t][1], zeros_tile)
        h, hw = _gru_step(h, parts, Wz, Uz, Wr, Ur, Wh, Uh, b3,
                          W_edge[(t + 1) % 4])

    y = _readout(h, x, Wa1, ba1.reshape(1, CC), Wb1, bb1.reshape(1, CC),
                 Wa2, ba2.reshape(1, CC), Wb2, bb2.reshape(1, CC),
                 fca_w, fca_b.reshape(1, NC), fcb_w, fcb_b.reshape(1, NC))
    return y.reshape(NC)


# speedup vs baseline: 1.2332x; 1.2332x over previous
import functools

import jax
import jax.numpy as jnp
from jax import lax
from jax.experimental import pallas as pl
from jax.experimental.pallas import tpu as pltpu
from jax.experimental.pallas import tpu_sc as plsc

N = 10000
E = 320000
D = 128
EMB = 128
F1 = D + EMB
CC = 128
NC = 2

NUM_CORES = 2
NUM_SUB = 16
NW = NUM_CORES * NUM_SUB
EPT = E // NW
K = 128
KB = 39
BLK_E = KB * K
KT = EPT - 2 * BLK_E
NP = 10240
ROWS_PER_TILE = NP // NUM_SUB
ZR = 160


@functools.lru_cache(maxsize=1)
def _build_sc_segment_sum():
    mesh = plsc.VectorSubcoreMesh(core_axis_name="c", subcore_axis_name="s",
                                  num_cores=NUM_CORES, num_subcores=NUM_SUB)

    @functools.partial(
        pl.kernel,
        out_type=jax.ShapeDtypeStruct((NUM_CORES, NP, D), jnp.float32),
        mesh=mesh,
        scratch_types=[
            pltpu.VMEM((BLK_E + KT,), jnp.int32),
            pltpu.VMEM((BLK_E + KT,), jnp.int32),
            pltpu.VMEM((K, D), jnp.float32),
            pltpu.VMEM((K, D), jnp.float32),
            pltpu.SemaphoreType.DMA,
            pltpu.SemaphoreType.DMA,
            pltpu.SemaphoreType.DMA,
            pltpu.VMEM_SHARED((NP, D), jnp.float32),
        ],
        compiler_params=pltpu.CompilerParams(use_tc_tiling_on_sc=False),
    )
    def _sc_segment_sum(hw_hbm, src_hbm, dst_hbm, zeros_hbm, out_hbm,
                        src_v, dst_v, rows0_v, rows1_v,
                        gsem0, gsem1, zsem, acc):
        c = lax.axis_index("c")
        s = lax.axis_index("s")
        w = s * NUM_CORES + c
        rows0 = s * ROWS_PER_TILE

        def sidx(j):
            return src_v.at[pl.ds(j * K, K)]

        def didx(j):
            return dst_v.at[pl.ds(j * K, K)]

        def gwait(rows_v, gsem, j):
            pltpu.make_async_copy(hw_hbm.at[sidx(j)], rows_v, gsem).wait()

        def scat(rows_v, j):
            pltpu.sync_copy(rows_v, acc.at[didx(j)], add=True)

        def body(jj, carry):
            j = jj * 2
            gwait(rows0_v, gsem0, j)
            scat(rows0_v, j)
            pltpu.async_copy(hw_hbm.at[sidx(j + 2)], rows0_v, gsem0)
            gwait(rows1_v, gsem1, j + 1)
            scat(rows1_v, j + 1)
            pltpu.async_copy(hw_hbm.at[sidx(j + 3)], rows1_v, gsem1)
            return carry

        for i in range(ROWS_PER_TILE // ZR):
            pltpu.async_copy(zeros_hbm, acc.at[pl.ds(rows0 + i * ZR, ZR)],
                             zsem)
        base = w * EPT
        pltpu.sync_copy(src_hbm.at[pl.ds(base, BLK_E)],
                        src_v.at[pl.ds(0, BLK_E)])
        pltpu.sync_copy(dst_hbm.at[pl.ds(base, BLK_E)],
                        dst_v.at[pl.ds(0, BLK_E)])
        pltpu.async_copy(hw_hbm.at[sidx(0)], rows0_v, gsem0)
        pltpu.async_copy(hw_hbm.at[sidx(1)], rows1_v, gsem1)
        for i in range(ROWS_PER_TILE // ZR):
            pltpu.make_async_copy(zeros_hbm, acc.at[pl.ds(rows0, ZR)],
                                  zsem).wait()
        plsc.subcore_barrier()

        lax.fori_loop(0, (KB - 3) // 2, body, 0)
        gwait(rows0_v, gsem0, KB - 3)
        scat(rows0_v, KB - 3)
        pltpu.async_copy(hw_hbm.at[sidx(KB - 1)], rows0_v, gsem0)
        gwait(rows1_v, gsem1, KB - 2)
        scat(rows1_v, KB - 2)
        gwait(rows0_v, gsem0, KB - 1)
        scat(rows0_v, KB - 1)

        pltpu.sync_copy(src_hbm.at[pl.ds(base + BLK_E, BLK_E + KT)], src_v)
        pltpu.sync_copy(dst_hbm.at[pl.ds(base + BLK_E, BLK_E + KT)], dst_v)
        pltpu.async_copy(hw_hbm.at[sidx(0)], rows0_v, gsem0)
        pltpu.async_copy(hw_hbm.at[sidx(1)], rows1_v, gsem1)
        lax.fori_loop(0, (KB - 3) // 2, body, 0)
        rt = rows1_v.at[pl.ds(0, KT)]
        sit = src_v.at[pl.ds(BLK_E, KT)]
        dit = dst_v.at[pl.ds(BLK_E, KT)]
        gwait(rows0_v, gsem0, KB - 3)
        scat(rows0_v, KB - 3)
        pltpu.async_copy(hw_hbm.at[sidx(KB - 1)], rows0_v, gsem0)
        gwait(rows1_v, gsem1, KB - 2)
        scat(rows1_v, KB - 2)
        pltpu.async_copy(hw_hbm.at[sit], rt, gsem1)
        gwait(rows0_v, gsem0, KB - 1)
        scat(rows0_v, KB - 1)
        pltpu.make_async_copy(hw_hbm.at[sit], rt, gsem1).wait()
        pltpu.sync_copy(rt, acc.at[dit], add=True)
        plsc.subcore_barrier()
        pltpu.sync_copy(acc.at[pl.ds(rows0, ROWS_PER_TILE)],
                        out_hbm.at[c, pl.ds(rows0, ROWS_PER_TILE)])

    def wrapped(hw, src, dst, zeros):
        return _sc_segment_sum(hw, src, dst, zeros)

    return wrapped


BR = 1000


def _mm_body(x_ref, w_ref, o_ref):
    o_ref[...] = jnp.dot(x_ref[...], w_ref[...],
                         preferred_element_type=jnp.float32)


_mm = pl.pallas_call(
    _mm_body,
    grid=(N // BR,),
    in_specs=[pl.BlockSpec((BR, D), lambda i: (i, 0)),
              pl.BlockSpec((D, D), lambda i: (0, 0))],
    out_specs=pl.BlockSpec((BR, D), lambda i: (i, 0)),
    out_shape=jax.ShapeDtypeStruct((N, D), jnp.float32),
)


def _gru_body(h_ref, p_ref, Wz_ref, Uz_ref, Wr_ref, Ur_ref,
              Wh_ref, Uh_ref, b_ref, Wn_ref, ho_ref, hwo_ref):
    h = h_ref[...]
    m = p_ref[0] + p_ref[1]
    dot = functools.partial(jnp.dot, preferred_element_type=jnp.float32)
    z = jax.nn.sigmoid(dot(m, Wz_ref[...]) + dot(h, Uz_ref[...]) + b_ref[0])
    r = jax.nn.sigmoid(dot(m, Wr_ref[...]) + dot(h, Ur_ref[...]) + b_ref[1])
    hc = jnp.tanh(dot(m, Wh_ref[...]) + dot(r * h, Uh_ref[...]) + b_ref[2])
    hn = (1.0 - z) * h + z * hc
    ho_ref[...] = hn
    hwo_ref[...] = dot(hn, Wn_ref[...])


_gru_step = pl.pallas_call(
    _gru_body,
    grid=(N // BR,),
    in_specs=[pl.BlockSpec((BR, D), lambda i: (i, 0)),
              pl.BlockSpec((2, BR, D), lambda i: (0, i, 0)),
              pl.BlockSpec((D, D), lambda i: (0, 0)),
              pl.BlockSpec((D, D), lambda i: (0, 0)),
              pl.BlockSpec((D, D), lambda i: (0, 0)),
              pl.BlockSpec((D, D), lambda i: (0, 0)),
              pl.BlockSpec((D, D), lambda i: (0, 0)),
              pl.BlockSpec((D, D), lambda i: (0, 0)),
              pl.BlockSpec((3, D), lambda i: (0, 0)),
              pl.BlockSpec((D, D), lambda i: (0, 0))],
    out_specs=[pl.BlockSpec((BR, D), lambda i: (i, 0)),
               pl.BlockSpec((BR, D), lambda i: (i, 0))],
    out_shape=[jax.ShapeDtypeStruct((N, D), jnp.float32),
               jax.ShapeDtypeStruct((N, D), jnp.float32)],
)


def _shift_down(a):
    return jnp.concatenate([jnp.zeros((1, a.shape[1]), a.dtype), a[:-1]], 0)


def _shift_up(a):
    return jnp.concatenate([a[1:], jnp.zeros((1, a.shape[1]), a.dtype)], 0)


def _conv3_in(a, W_ref, b_ref):
    dot = functools.partial(jnp.dot, preferred_element_type=jnp.float32)
    return (dot(_shift_down(a), W_ref[0]) + dot(a, W_ref[1])
            + dot(_shift_up(a), W_ref[2]) + b_ref[0])


def _pool3s2(c):
    r = c.reshape(c.shape[0] // 2, 2, c.shape[1])
    e = r[:, 0, :]
    o = r[:, 1, :]
    return jnp.maximum(jnp.maximum(e, o), _shift_down(o))


def _readout_body(h_ref, x_ref, Wa1_ref, ba1_ref, Wb1_ref, bb1_ref,
                  Wa2_ref, ba2_ref, Wb2_ref, bb2_ref,
                  wa_ref, fa_ref, wb_ref, fb_ref, o_ref):
    dot = functools.partial(jnp.dot, preferred_element_type=jnp.float32)
    h = h_ref[...]
    zc = jnp.concatenate([h, x_ref[...]], 1)
    pa = _pool3s2(jnp.maximum(_conv3_in(zc, Wa1_ref, ba1_ref), 0.0))
    pb = _pool3s2(jnp.maximum(_conv3_in(h, Wb1_ref, bb1_ref), 0.0))
    p2a = _pool3s2(jnp.maximum(_conv3_in(pa, Wa2_ref, ba2_ref), 0.0))
    p2b = _pool3s2(jnp.maximum(_conv3_in(pb, Wb2_ref, bb2_ref), 0.0))
    ya = dot(p2a, wa_ref[...]) + fa_ref[0]
    yb = dot(p2b, wb_ref[...]) + fb_ref[0]
    o_ref[...] = jnp.sum(ya * yb, axis=0, keepdims=True) * (1.0 / (N // 4))


_readout = pl.pallas_call(
    _readout_body,
    out_shape=jax.ShapeDtypeStruct((1, NC), jnp.float32),
    compiler_params=pltpu.CompilerParams(vmem_limit_bytes=100 * 1024 * 1024),
)


def kernel(x, ast_edge_index, cfg_edge_index, ddg_edge_index, ncs_edge_index,
           W_edge, Wz, Uz, bz, Wr, Ur, br, Wh, Uh, bh,
           Wa1, ba1, Wa2, ba2, fca_w, fca_b,
           Wb1, bb1, Wb2, bb2, fcb_w, fcb_b):
    edges = [ast_edge_index, cfg_edge_index, ddg_edge_index, ncs_edge_index]
    zeros_tile = jnp.zeros((ZR, D), jnp.float32)
    b3 = jnp.stack([bz, br, bh])

    sc_segment_sum = _build_sc_segment_sum()
    h = x
    hw = _mm(h, W_edge[0])
    for t in range(4):
        parts = sc_segment_sum(hw, edges[t][0], edges[t][1], zeros_tile)
        h, hw = _gru_step(h, parts, Wz, Uz, Wr, Ur, Wh, Uh, b3,
                          W_edge[(t + 1) % 4])

    y = _readout(h, x, Wa1, ba1.reshape(1, CC), Wb1, bb1.reshape(1, CC),
                 Wa2, ba2.reshape(1, CC), Wb2, bb2.reshape(1, CC),
                 fca_w, fca_b.reshape(1, NC), fcb_w, fcb_b.reshape(1, NC))
    return y.reshape(NC)
